# baseline (speedup 1.0000x reference)
import numpy as np

import jax
import jax.numpy as jnp
from jax import lax
from jax.experimental import pallas as pl
from jax.experimental.pallas import tpu as pltpu
from jax.experimental.pallas import tpu_sc as plsc

N = 10000
D = 128
E = 320000
ED = 16
NG = 64
L = 2

NC = 2
NS = 16
NW = NC * NS
EW = E // NW
EB = 40
CH = 1000
CB = CH // EB
NCH = EW // CH
N2 = 10000
RPT = N2 // NS
ZR = 25
QW = D + 32
KW = 2 * D
MW = 160
INV_SQRT_D = 1.0 / float(D) ** 0.5

f32 = jnp.float32
bf16 = jnp.bfloat16
i32 = jnp.int32

_RDUP = np.zeros((ED, 32), np.float32)
for _j in range(ED):
    _RDUP[_j, 2 * _j] = 1.0
    _RDUP[_j, 2 * _j + 1] = 1.0

_PPRE = np.zeros((D, D), np.float32)
for _c in range(4):
    for _j in range(16):
        _PPRE[32 * _c + _j, 32 * _c + 2 * _j] = 1.0
        _PPRE[32 * _c + 16 + _j, 32 * _c + 2 * _j + 1] = 1.0


def _edge_body(q2_h, kv_h, ea_h, srcb_h, dstb_h,
               acc_h,
               dstb0, dstb1, srcb0, srcb1, q2r0, q2r1, kvr0, kvr1,
               ear0, ear1, msg, acc_s,
               semq0, semq1, semk0, semk1, seme0, seme1, semsc,
               semi0, semi1):
    cid = lax.axis_index("c")
    sid = lax.axis_index("s")
    wid = sid * NC + cid
    z16 = jnp.zeros((16,), f32)
    onehot0 = jnp.where(lax.iota(i32, 16) == 0, 1.0, 0.0).astype(f32)

    dstbs = (dstb0, dstb1)
    srcbs = (srcb0, srcb1)
    semi = (semi0, semi1)
    q2r = (q2r0, q2r1)
    kvr = (kvr0, kvr1)
    ear = (ear0, ear1)
    semq = (semq0, semq1)
    semk = (semk0, semk1)
    seme = (seme0, seme1)

    @pl.loop(0, ZR)
    def _zfill(r):
        for c in range(MW // 16):
            msg[r, pl.ds(16 * c, 16)] = z16

    @pl.loop(0, RPT // ZR)
    def _zinit(j):
        pltpu.async_copy(msg.at[pl.ds(0, ZR)],
                         acc_s.at[pl.ds(sid * RPT + j * ZR, ZR)], semsc)

    @pl.loop(0, RPT // ZR)
    def _zdrain(j):
        pltpu.make_async_copy(msg.at[pl.ds(0, ZR)],
                              acc_s.at[pl.ds(sid * RPT + j * ZR, ZR)],
                              semsc).wait()

    plsc.subcore_barrier()

    def idx_fetch(ch, pc):
        row0 = wid * (EW // EB) + ch * CB
        pltpu.async_copy(dstb_h.at[pl.ds(row0, CB)], dstbs[pc], semi[pc])
        pltpu.async_copy(srcb_h.at[pl.ds(row0, CB)], srcbs[pc], semi[pc])

    def idx_wait(ch, pc):
        row0 = wid * (EW // EB) + ch * CB
        pltpu.make_async_copy(dstb_h.at[pl.ds(row0, CB)], dstbs[pc], semi[pc]).wait()
        pltpu.make_async_copy(srcb_h.at[pl.ds(row0, CB)], srcbs[pc], semi[pc]).wait()

    def issue(dstb, srcb, b, p, ebase_ch):
        pltpu.async_copy(q2_h.at[dstb.at[b]], q2r[p], semq[p])
        pltpu.async_copy(kv_h.at[srcb.at[b]], kvr[p], semk[p])
        pltpu.async_copy(ea_h.at[pl.ds(ebase_ch + b * EB, EB)], ear[p], seme[p])

    unpack = plsc.unpack
    ILV = plsc.PackFormat.INTERLEAVED

    def compute(p):
        q2b, kvb, eab = q2r[p], kvr[p], ear[p]
        for u in range(EB):
            ea_u = eab[u]
            qe_a, _ = unpack(q2b[u, pl.ds(D, 32)], format=ILV)
            acc = qe_a * ea_u
            for c in range(D // 32):
                qa, qb = unpack(q2b[u, pl.ds(32 * c, 32)], format=ILV)
                ka, kb = unpack(kvb[u, pl.ds(32 * c, 32)], format=ILV)
                acc = acc + qa * ka + qb * kb
            s = jnp.sum(acc) * INV_SQRT_D
            ex = jnp.exp(jnp.broadcast_to(s, (16,)))
            for c in range(D // 32):
                va, vb = unpack(kvb[u, pl.ds(D + 32 * c, 32)], format=ILV)
                msg[u, pl.ds(32 * c, 16)] = va * ex
                msg[u, pl.ds(32 * c + 16, 16)] = vb * ex
            msg[u, pl.ds(D, 16)] = ea_u * ex
            msg[u, pl.ds(D + 16, 16)] = ex * onehot0

    idx_fetch(0, 0)

    @pl.loop(0, NCH)
    def _chunk(ch):
        ebase_ch = wid * EW + ch * CH
        for pc in (0, 1):
            @pl.when(ch % 2 == pc)
            def _():
                dstb = dstbs[pc]
                srcb = srcbs[pc]
                idx_wait(ch, pc)

                @pl.when(ch + 1 < NCH)
                def _():
                    idx_fetch(ch + 1, 1 - pc)

                issue(dstb, srcb, 0, 0, ebase_ch)

                @pl.loop(0, CB)
                def _blk(b):
                    for p in (0, 1):
                        @pl.when(b % 2 == p)
                        def _():
                            @pl.when(b + 1 < CB)
                            def _():
                                issue(dstb, srcb, b + 1, 1 - p, ebase_ch)

                            pltpu.make_async_copy(q2_h.at[dstb.at[b]], q2r[p], semq[p]).wait()
                            pltpu.make_async_copy(kv_h.at[srcb.at[b]], kvr[p], semk[p]).wait()
                            pltpu.make_async_copy(
                                ea_h.at[pl.ds(ebase_ch + b * EB, EB)], ear[p], seme[p]).wait()

                            @pl.when(b >= 1)
                            def _():
                                pltpu.make_async_copy(
                                    msg, acc_s.at[dstb.at[b - 1]], semsc).wait()

                            compute(p)
                            pltpu.async_copy(msg, acc_s.at[dstb.at[b]], semsc, add=True)

                pltpu.make_async_copy(msg, acc_s.at[dstb.at[CB - 1]], semsc).wait()

    plsc.subcore_barrier()
    out_base = sid * RPT
    pltpu.sync_copy(acc_s.at[pl.ds(out_base, RPT)],
                    acc_h.at[cid, pl.ds(out_base, RPT)])


_edge_call = pl.kernel(
    _edge_body,
    out_type=[jax.ShapeDtypeStruct((NC, N2, MW), f32)],
    mesh=plsc.VectorSubcoreMesh(core_axis_name="c", subcore_axis_name="s",
                                num_cores=NC, num_subcores=NS),
    compiler_params=pltpu.CompilerParams(needs_layout_passes=False,
                                         use_tc_tiling_on_sc=False),
    scratch_types=[
        pltpu.VMEM((CB, EB), i32),
        pltpu.VMEM((CB, EB), i32),
        pltpu.VMEM((CB, EB), i32),
        pltpu.VMEM((CB, EB), i32),
        pltpu.VMEM((EB, QW), bf16),
        pltpu.VMEM((EB, QW), bf16),
        pltpu.VMEM((EB, KW), bf16),
        pltpu.VMEM((EB, KW), bf16),
        pltpu.VMEM((EB, ED), f32),
        pltpu.VMEM((EB, ED), f32),
        pltpu.VMEM((EB, MW), f32),
        pltpu.VMEM_SHARED((N2, MW), f32),
        pltpu.SemaphoreType.DMA, pltpu.SemaphoreType.DMA,
        pltpu.SemaphoreType.DMA, pltpu.SemaphoreType.DMA,
        pltpu.SemaphoreType.DMA, pltpu.SemaphoreType.DMA,
        pltpu.SemaphoreType.DMA, pltpu.SemaphoreType.DMA,
        pltpu.SemaphoreType.DMA,
    ],
)


RB = 400
NRB = N // RB


def _pre_body(h_ref, wq_ref, wk_ref, wvp_ref, wet2_ref, bq_ref, bke_ref,
              bvp_ref, q2_ref, kv_ref):
    h = h_ref[...]
    q = jnp.dot(h, wq_ref[...], preferred_element_type=f32) + bq_ref[...]
    q2_ref[:, 0:D] = q.astype(bf16)
    q2_ref[:, D:QW] = jnp.dot(q, wet2_ref[...], preferred_element_type=f32).astype(bf16)
    kv_ref[:, 0:D] = (jnp.dot(h, wk_ref[...], preferred_element_type=f32)
                      + bke_ref[...]).astype(bf16)
    kv_ref[:, D:KW] = (jnp.dot(h, wvp_ref[...], preferred_element_type=f32)
                       + bvp_ref[...]).astype(bf16)


_pre_call = pl.pallas_call(
    _pre_body,
    grid=(NRB,),
    in_specs=[
        pl.BlockSpec((RB, D), lambda i: (i, 0)),
        pl.BlockSpec((D, D), lambda i: (0, 0)),
        pl.BlockSpec((D, D), lambda i: (0, 0)),
        pl.BlockSpec((D, D), lambda i: (0, 0)),
        pl.BlockSpec((D, 32), lambda i: (0, 0)),
        pl.BlockSpec((1, D), lambda i: (0, 0)),
        pl.BlockSpec((1, D), lambda i: (0, 0)),
        pl.BlockSpec((1, D), lambda i: (0, 0)),
    ],
    out_specs=[
        pl.BlockSpec((RB, QW), lambda i: (i, 0)),
        pl.BlockSpec((RB, KW), lambda i: (i, 0)),
    ],
    out_shape=[
        jax.ShapeDtypeStruct((N, QW), bf16),
        jax.ShapeDtypeStruct((N, KW), bf16),
    ],
)


def _post_body(acc_ref, h_ref, we_ref, wskip_ref, bskip_ref,
               wbeta_ref, wt_ref, bt_ref, h2_ref, ssum_ref, ssq_ref):
    acc = acc_ref[0] + acc_ref[1]
    accv = acc[:, 0:D]
    ea16 = acc[:, D:D + ED]
    den = acc[:, D + ED:D + ED + 1]
    out = (accv + jnp.dot(ea16, we_ref[...], preferred_element_type=f32)) / (den + 1e-16)
    r = jnp.dot(h_ref[...], wskip_ref[...], preferred_element_type=f32) + bskip_ref[...]
    wb = wbeta_ref[...]
    blog = (jnp.dot(out, wb[0:D], preferred_element_type=f32)
            + jnp.dot(r, wb[D:2 * D], preferred_element_type=f32)
            + jnp.dot(out - r, wb[2 * D:3 * D], preferred_element_type=f32))
    beta = jax.nn.sigmoid(blog)
    h2 = beta * r + (1.0 - beta) * out
    h2 = jnp.maximum(jnp.dot(h2, wt_ref[...], preferred_element_type=f32) + bt_ref[...], 0.0)
    h2_ref[...] = h2

    @pl.when(pl.program_id(0) == 0)
    def _():
        ssum_ref[...] = jnp.zeros_like(ssum_ref)
        ssq_ref[...] = jnp.zeros_like(ssq_ref)

    ssum_ref[...] += jnp.sum(h2, axis=0, keepdims=True)
    ssq_ref[...] += jnp.sum(h2 * h2, axis=0, keepdims=True)


_post_call = pl.pallas_call(
    _post_body,
    grid=(NRB,),
    in_specs=[
        pl.BlockSpec((NC, RB, MW), lambda i: (0, i, 0)),
        pl.BlockSpec((RB, D), lambda i: (i, 0)),
        pl.BlockSpec((ED, D), lambda i: (0, 0)),
        pl.BlockSpec((D, D), lambda i: (0, 0)),
        pl.BlockSpec((1, D), lambda i: (0, 0)),
        pl.BlockSpec((3 * D, 1), lambda i: (0, 0)),
        pl.BlockSpec((D, D), lambda i: (0, 0)),
        pl.BlockSpec((1, D), lambda i: (0, 0)),
    ],
    out_specs=[
        pl.BlockSpec((RB, D), lambda i: (i, 0)),
        pl.BlockSpec((1, D), lambda i: (0, 0)),
        pl.BlockSpec((1, D), lambda i: (0, 0)),
    ],
    out_shape=[
        jax.ShapeDtypeStruct((N, D), f32),
        jax.ShapeDtypeStruct((1, D), f32),
        jax.ShapeDtypeStruct((1, D), f32),
    ],
)


def _bn_body(h2_ref, ssum_ref, ssq_ref, gamma_ref, bbeta_ref, out_ref):
    mu = ssum_ref[...] * (1.0 / N)
    var = ssq_ref[...] * (1.0 / N) - mu * mu
    scale = gamma_ref[...] / jnp.sqrt(var + 1e-5)
    out_ref[...] = (h2_ref[...] - mu) * scale + bbeta_ref[...]


_bn_call = pl.pallas_call(
    _bn_body,
    grid=(NRB,),
    in_specs=[
        pl.BlockSpec((RB, D), lambda i: (i, 0)),
        pl.BlockSpec((1, D), lambda i: (0, 0)),
        pl.BlockSpec((1, D), lambda i: (0, 0)),
        pl.BlockSpec((1, D), lambda i: (0, 0)),
        pl.BlockSpec((1, D), lambda i: (0, 0)),
    ],
    out_specs=pl.BlockSpec((RB, D), lambda i: (i, 0)),
    out_shape=jax.ShapeDtypeStruct((N, D), f32),
)


def _pool_body(h0_ref, h1_ref, bi_ref, wg_ref, bg_ref, w1_ref, b1_ref,
               w2_ref, b2_ref, z_ref, pnum_ref, pden_ref):
    hs = h0_ref[...] + h1_ref[...]
    g = jnp.dot(hs, wg_ref[...], preferred_element_type=f32) + bg_ref[...]
    ge = jnp.exp(g)
    bi = bi_ref[0]
    gid = lax.broadcasted_iota(i32, (NG, RB), 0).astype(f32)
    mt = jnp.where(gid == bi, 1.0, 0.0)

    @pl.when(pl.program_id(0) == 0)
    def _():
        pnum_ref[...] = jnp.zeros_like(pnum_ref)
        pden_ref[...] = jnp.zeros_like(pden_ref)

    pnum_ref[...] += jnp.dot(mt, hs * ge, preferred_element_type=f32)
    pden_ref[...] += jnp.dot(mt, ge, preferred_element_type=f32)

    @pl.when(pl.program_id(0) == NRB - 1)
    def _():
        pooled = pnum_ref[...] / (pden_ref[...] + 1e-16)
        z1 = jnp.maximum(jnp.dot(pooled, w1_ref[...], preferred_element_type=f32)
                         + b1_ref[...], 0.0)
        z_ref[...] = jnp.dot(z1, w2_ref[...], preferred_element_type=f32) + b2_ref[...]


_pool_call = pl.pallas_call(
    _pool_body,
    grid=(NRB,),
    in_specs=[
        pl.BlockSpec((RB, D), lambda i: (i, 0)),
        pl.BlockSpec((RB, D), lambda i: (i, 0)),
        pl.BlockSpec((1, 1, RB), lambda i: (i, 0, 0)),
        pl.BlockSpec((D, 1), lambda i: (0, 0)),
        pl.BlockSpec((1, 1), lambda i: (0, 0)),
        pl.BlockSpec((D, D // 2), lambda i: (0, 0)),
        pl.BlockSpec((1, D // 2), lambda i: (0, 0)),
        pl.BlockSpec((D // 2, 1), lambda i: (0, 0)),
        pl.BlockSpec((1, 1), lambda i: (0, 0)),
    ],
    out_specs=[
        pl.BlockSpec((NG, 1), lambda i: (0, 0)),
        pl.BlockSpec((NG, D), lambda i: (0, 0)),
        pl.BlockSpec((NG, 1), lambda i: (0, 0)),
    ],
    out_shape=[
        jax.ShapeDtypeStruct((NG, 1), f32),
        jax.ShapeDtypeStruct((NG, D), f32),
        jax.ShapeDtypeStruct((NG, 1), f32),
    ],
)


def kernel(x, edge_index, edge_attr, batch_index, Wq, bq, Wk, bk, Wv, bv,
           We, be, Wskip, bskip, Wbeta, Wt, bt, bn_gamma, bn_beta, Wg, bg,
           W1, b1, W2, b2):
    srcb = edge_index[0].astype(i32).reshape(E // EB, EB)
    dstb = edge_index[1].astype(i32).reshape(E // EB, EB)
    ea = edge_attr.astype(f32)
    bi = batch_index.astype(f32).reshape(NRB, 1, RB)
    rdup = jnp.asarray(_RDUP)
    ppre = jnp.asarray(_PPRE)

    h = x
    locs = []
    for l in range(L):
        wet2 = We[l].T @ rdup
        wvp = Wv[l] @ ppre
        bvp = ((bv[l] + be[l]).reshape(1, D)) @ ppre
        bke = (bk[l] + be[l]).reshape(1, D)
        q2, kv = _pre_call(h, Wq[l], Wk[l], wvp, wet2,
                           bq[l].reshape(1, D), bke, bvp)
        (acc,) = _edge_call(q2, kv, ea, srcb, dstb)
        h2pre, ssum, ssq = _post_call(
            acc, h, We[l], Wskip[l], bskip[l].reshape(1, D),
            Wbeta[l], Wt[l], bt[l].reshape(1, D))
        h = _bn_call(h2pre, ssum, ssq,
                     bn_gamma[l].reshape(1, D), bn_beta[l].reshape(1, D))
        locs.append(h)

    z, _, _ = _pool_call(
        locs[0], locs[1], bi, Wg, bg.reshape(1, 1), W1, b1.reshape(1, D // 2),
        W2, b2.reshape(1, 1))
    return z


# speedup vs baseline: 1.0996x; 1.0996x over previous
import numpy as np

import jax
import jax.numpy as jnp
from jax import lax
from jax.experimental import pallas as pl
from jax.experimental.pallas import tpu as pltpu
from jax.experimental.pallas import tpu_sc as plsc

N = 10000
D = 128
E = 320000
ED = 16
NG = 64
L = 2

NC = 2
NS = 16
NW = NC * NS
EW = E // NW
EB = 40
CH = 1000
CB = CH // EB
NCH = EW // CH
N2 = 10000
RPT = N2 // NS
ZR = 25
QW = D + 32
KW = 2 * D
MW = 160
INV_SQRT_D = 1.0 / float(D) ** 0.5

f32 = jnp.float32
bf16 = jnp.bfloat16
i32 = jnp.int32

_RDUP = np.zeros((ED, 32), np.float32)
for _j in range(ED):
    _RDUP[_j, 2 * _j] = 1.0
    _RDUP[_j, 2 * _j + 1] = 1.0

_PPRE = np.zeros((D, D), np.float32)
for _c in range(4):
    for _j in range(16):
        _PPRE[32 * _c + _j, 32 * _c + 2 * _j] = 1.0
        _PPRE[32 * _c + 16 + _j, 32 * _c + 2 * _j + 1] = 1.0


def _edge_body(q2_h, kv_h, ea_h, srcb_h, dstb_h,
               acc_h,
               dstb, srcb, q2r0, q2r1, kvr0, kvr1, ear0, ear1, msg, acc_s,
               semq0, semq1, semk0, semk1, seme0, seme1, semsc):
    cid = lax.axis_index("c")
    sid = lax.axis_index("s")
    wid = sid * NC + cid
    z16 = jnp.zeros((16,), f32)
    onehot0 = jnp.where(lax.iota(i32, 16) == 0, 1.0, 0.0).astype(f32)

    q2r = (q2r0, q2r1)
    kvr = (kvr0, kvr1)
    ear = (ear0, ear1)
    semq = (semq0, semq1)
    semk = (semk0, semk1)
    seme = (seme0, seme1)

    @pl.loop(0, ZR)
    def _zfill(r):
        for c in range(MW // 16):
            msg[r, pl.ds(16 * c, 16)] = z16

    @pl.loop(0, RPT // ZR)
    def _zinit(j):
        pltpu.sync_copy(msg.at[pl.ds(0, ZR)],
                        acc_s.at[pl.ds(sid * RPT + j * ZR, ZR)])

    plsc.subcore_barrier()

    def issue(b, p, ebase_ch):
        pltpu.async_copy(q2_h.at[dstb.at[b]], q2r[p], semq[p])
        pltpu.async_copy(kv_h.at[srcb.at[b]], kvr[p], semk[p])
        pltpu.async_copy(ea_h.at[pl.ds(ebase_ch + b * EB, EB)], ear[p], seme[p])

    unpack = plsc.unpack
    ILV = plsc.PackFormat.INTERLEAVED

    def compute(p):
        q2b, kvb, eab = q2r[p], kvr[p], ear[p]
        for u in range(EB):
            ea_u = eab[u]
            qe_a, _ = unpack(q2b[u, pl.ds(D, 32)], format=ILV)
            acc = qe_a * ea_u
            for c in range(D // 32):
                qa, qb = unpack(q2b[u, pl.ds(32 * c, 32)], format=ILV)
                ka, kb = unpack(kvb[u, pl.ds(32 * c, 32)], format=ILV)
                acc = acc + qa * ka + qb * kb
            s = jnp.sum(acc) * INV_SQRT_D
            ex = jnp.exp(jnp.broadcast_to(s, (16,)))
            for c in range(D // 32):
                va, vb = unpack(kvb[u, pl.ds(D + 32 * c, 32)], format=ILV)
                msg[u, pl.ds(32 * c, 16)] = va * ex
                msg[u, pl.ds(32 * c + 16, 16)] = vb * ex
            msg[u, pl.ds(D, 16)] = ea_u * ex
            msg[u, pl.ds(D + 16, 16)] = ex * onehot0

    @pl.loop(0, NCH)
    def _chunk(ch):
        row0 = wid * (EW // EB) + ch * CB
        ebase_ch = wid * EW + ch * CH
        pltpu.sync_copy(dstb_h.at[pl.ds(row0, CB)], dstb)
        pltpu.sync_copy(srcb_h.at[pl.ds(row0, CB)], srcb)
        issue(0, 0, ebase_ch)

        @pl.loop(0, CB)
        def _blk(b):
            for p in (0, 1):
                @pl.when(b % 2 == p)
                def _():
                    @pl.when(b + 1 < CB)
                    def _():
                        issue(b + 1, 1 - p, ebase_ch)

                    pltpu.make_async_copy(q2_h.at[dstb.at[b]], q2r[p], semq[p]).wait()
                    pltpu.make_async_copy(kv_h.at[srcb.at[b]], kvr[p], semk[p]).wait()
                    pltpu.make_async_copy(
                        ea_h.at[pl.ds(ebase_ch + b * EB, EB)], ear[p], seme[p]).wait()

                    @pl.when(b >= 1)
                    def _():
                        pltpu.make_async_copy(
                            msg, acc_s.at[dstb.at[b - 1]], semsc).wait()

                    compute(p)
                    pltpu.async_copy(msg, acc_s.at[dstb.at[b]], semsc, add=True)

        pltpu.make_async_copy(msg, acc_s.at[dstb.at[CB - 1]], semsc).wait()

    plsc.subcore_barrier()
    out_base = sid * RPT
    pltpu.sync_copy(acc_s.at[pl.ds(out_base, RPT)],
                    acc_h.at[cid, pl.ds(out_base, RPT)])


_edge_call = pl.kernel(
    _edge_body,
    out_type=[jax.ShapeDtypeStruct((NC, N2, MW), f32)],
    mesh=plsc.VectorSubcoreMesh(core_axis_name="c", subcore_axis_name="s",
                                num_cores=NC, num_subcores=NS),
    compiler_params=pltpu.CompilerParams(needs_layout_passes=False,
                                         use_tc_tiling_on_sc=False),
    scratch_types=[
        pltpu.VMEM((CB, EB), i32),
        pltpu.VMEM((CB, EB), i32),
        pltpu.VMEM((EB, QW), bf16),
        pltpu.VMEM((EB, QW), bf16),
        pltpu.VMEM((EB, KW), bf16),
        pltpu.VMEM((EB, KW), bf16),
        pltpu.VMEM((EB, ED), f32),
        pltpu.VMEM((EB, ED), f32),
        pltpu.VMEM((EB, MW), f32),
        pltpu.VMEM_SHARED((N2, MW), f32),
        pltpu.SemaphoreType.DMA, pltpu.SemaphoreType.DMA,
        pltpu.SemaphoreType.DMA, pltpu.SemaphoreType.DMA,
        pltpu.SemaphoreType.DMA, pltpu.SemaphoreType.DMA,
        pltpu.SemaphoreType.DMA,
    ],
)


RB = 400
NRB = N // RB


def _pre_body(h_ref, wq_ref, wk_ref, wvp_ref, wet2_ref, bq_ref, bke_ref,
              bvp_ref, q2_ref, kv_ref):
    h = h_ref[...]
    q = jnp.dot(h, wq_ref[...], preferred_element_type=f32) + bq_ref[...]
    q2_ref[:, 0:D] = q.astype(bf16)
    q2_ref[:, D:QW] = jnp.dot(q, wet2_ref[...], preferred_element_type=f32).astype(bf16)
    kv_ref[:, 0:D] = (jnp.dot(h, wk_ref[...], preferred_element_type=f32)
                      + bke_ref[...]).astype(bf16)
    kv_ref[:, D:KW] = (jnp.dot(h, wvp_ref[...], preferred_element_type=f32)
                       + bvp_ref[...]).astype(bf16)


_pre_call = pl.pallas_call(
    _pre_body,
    grid=(NRB,),
    in_specs=[
        pl.BlockSpec((RB, D), lambda i: (i, 0)),
        pl.BlockSpec((D, D), lambda i: (0, 0)),
        pl.BlockSpec((D, D), lambda i: (0, 0)),
        pl.BlockSpec((D, D), lambda i: (0, 0)),
        pl.BlockSpec((D, 32), lambda i: (0, 0)),
        pl.BlockSpec((1, D), lambda i: (0, 0)),
        pl.BlockSpec((1, D), lambda i: (0, 0)),
        pl.BlockSpec((1, D), lambda i: (0, 0)),
    ],
    out_specs=[
        pl.BlockSpec((RB, QW), lambda i: (i, 0)),
        pl.BlockSpec((RB, KW), lambda i: (i, 0)),
    ],
    out_shape=[
        jax.ShapeDtypeStruct((N, QW), bf16),
        jax.ShapeDtypeStruct((N, KW), bf16),
    ],
)


def _post_body(acc_ref, h_ref, we_ref, wskip_ref, bskip_ref,
               wbeta_ref, wt_ref, bt_ref, h2_ref, ssum_ref, ssq_ref):
    acc = acc_ref[0] + acc_ref[1]
    accv = acc[:, 0:D]
    ea16 = acc[:, D:D + ED]
    den = acc[:, D + ED:D + ED + 1]
    out = (accv + jnp.dot(ea16, we_ref[...], preferred_element_type=f32)) / (den + 1e-16)
    r = jnp.dot(h_ref[...], wskip_ref[...], preferred_element_type=f32) + bskip_ref[...]
    wb = wbeta_ref[...]
    blog = (jnp.dot(out, wb[0:D], preferred_element_type=f32)
            + jnp.dot(r, wb[D:2 * D], preferred_element_type=f32)
            + jnp.dot(out - r, wb[2 * D:3 * D], preferred_element_type=f32))
    beta = jax.nn.sigmoid(blog)
    h2 = beta * r + (1.0 - beta) * out
    h2 = jnp.maximum(jnp.dot(h2, wt_ref[...], preferred_element_type=f32) + bt_ref[...], 0.0)
    h2_ref[...] = h2

    @pl.when(pl.program_id(0) == 0)
    def _():
        ssum_ref[...] = jnp.zeros_like(ssum_ref)
        ssq_ref[...] = jnp.zeros_like(ssq_ref)

    ssum_ref[...] += jnp.sum(h2, axis=0, keepdims=True)
    ssq_ref[...] += jnp.sum(h2 * h2, axis=0, keepdims=True)


_post_call = pl.pallas_call(
    _post_body,
    grid=(NRB,),
    in_specs=[
        pl.BlockSpec((NC, RB, MW), lambda i: (0, i, 0)),
        pl.BlockSpec((RB, D), lambda i: (i, 0)),
        pl.BlockSpec((ED, D), lambda i: (0, 0)),
        pl.BlockSpec((D, D), lambda i: (0, 0)),
        pl.BlockSpec((1, D), lambda i: (0, 0)),
        pl.BlockSpec((3 * D, 1), lambda i: (0, 0)),
        pl.BlockSpec((D, D), lambda i: (0, 0)),
        pl.BlockSpec((1, D), lambda i: (0, 0)),
    ],
    out_specs=[
        pl.BlockSpec((RB, D), lambda i: (i, 0)),
        pl.BlockSpec((1, D), lambda i: (0, 0)),
        pl.BlockSpec((1, D), lambda i: (0, 0)),
    ],
    out_shape=[
        jax.ShapeDtypeStruct((N, D), f32),
        jax.ShapeDtypeStruct((1, D), f32),
        jax.ShapeDtypeStruct((1, D), f32),
    ],
)


def _bn_body(h2_ref, ssum_ref, ssq_ref, gamma_ref, bbeta_ref, out_ref):
    mu = ssum_ref[...] * (1.0 / N)
    var = ssq_ref[...] * (1.0 / N) - mu * mu
    scale = gamma_ref[...] / jnp.sqrt(var + 1e-5)
    out_ref[...] = (h2_ref[...] - mu) * scale + bbeta_ref[...]


_bn_call = pl.pallas_call(
    _bn_body,
    grid=(NRB,),
    in_specs=[
        pl.BlockSpec((RB, D), lambda i: (i, 0)),
        pl.BlockSpec((1, D), lambda i: (0, 0)),
        pl.BlockSpec((1, D), lambda i: (0, 0)),
        pl.BlockSpec((1, D), lambda i: (0, 0)),
        pl.BlockSpec((1, D), lambda i: (0, 0)),
    ],
    out_specs=pl.BlockSpec((RB, D), lambda i: (i, 0)),
    out_shape=jax.ShapeDtypeStruct((N, D), f32),
)


def _pool_body(h0_ref, h1_ref, bi_ref, wg_ref, bg_ref, w1_ref, b1_ref,
               w2_ref, b2_ref, z_ref, pnum_ref, pden_ref):
    hs = h0_ref[...] + h1_ref[...]
    g = jnp.dot(hs, wg_ref[...], preferred_element_type=f32) + bg_ref[...]
    ge = jnp.exp(g)
    bi = bi_ref[0]
    gid = lax.broadcasted_iota(i32, (NG, RB), 0).astype(f32)
    mt = jnp.where(gid == bi, 1.0, 0.0)

    @pl.when(pl.program_id(0) == 0)
    def _():
        pnum_ref[...] = jnp.zeros_like(pnum_ref)
        pden_ref[...] = jnp.zeros_like(pden_ref)

    pnum_ref[...] += jnp.dot(mt, hs * ge, preferred_element_type=f32)
    pden_ref[...] += jnp.dot(mt, ge, preferred_element_type=f32)

    @pl.when(pl.program_id(0) == NRB - 1)
    def _():
        pooled = pnum_ref[...] / (pden_ref[...] + 1e-16)
        z1 = jnp.maximum(jnp.dot(pooled, w1_ref[...], preferred_element_type=f32)
                         + b1_ref[...], 0.0)
        z_ref[...] = jnp.dot(z1, w2_ref[...], preferred_element_type=f32) + b2_ref[...]


_pool_call = pl.pallas_call(
    _pool_body,
    grid=(NRB,),
    in_specs=[
        pl.BlockSpec((RB, D), lambda i: (i, 0)),
        pl.BlockSpec((RB, D), lambda i: (i, 0)),
        pl.BlockSpec((1, 1, RB), lambda i: (i, 0, 0)),
        pl.BlockSpec((D, 1), lambda i: (0, 0)),
        pl.BlockSpec((1, 1), lambda i: (0, 0)),
        pl.BlockSpec((D, D // 2), lambda i: (0, 0)),
        pl.BlockSpec((1, D // 2), lambda i: (0, 0)),
        pl.BlockSpec((D // 2, 1), lambda i: (0, 0)),
        pl.BlockSpec((1, 1), lambda i: (0, 0)),
    ],
    out_specs=[
        pl.BlockSpec((NG, 1), lambda i: (0, 0)),
        pl.BlockSpec((NG, D), lambda i: (0, 0)),
        pl.BlockSpec((NG, 1), lambda i: (0, 0)),
    ],
    out_shape=[
        jax.ShapeDtypeStruct((NG, 1), f32),
        jax.ShapeDtypeStruct((NG, D), f32),
        jax.ShapeDtypeStruct((NG, 1), f32),
    ],
)


def kernel(x, edge_index, edge_attr, batch_index, Wq, bq, Wk, bk, Wv, bv,
           We, be, Wskip, bskip, Wbeta, Wt, bt, bn_gamma, bn_beta, Wg, bg,
           W1, b1, W2, b2):
    srcb = edge_index[0].astype(i32).reshape(E // EB, EB)
    dstb = edge_index[1].astype(i32).reshape(E // EB, EB)
    ea = edge_attr.astype(f32)
    bi = batch_index.astype(f32).reshape(NRB, 1, RB)
    rdup = jnp.asarray(_RDUP)
    ppre = jnp.asarray(_PPRE)

    h = x
    locs = []
    for l in range(L):
        wet2 = We[l].T @ rdup
        wvp = Wv[l] @ ppre
        bvp = ((bv[l] + be[l]).reshape(1, D)) @ ppre
        bke = (bk[l] + be[l]).reshape(1, D)
        q2, kv = _pre_call(h, Wq[l], Wk[l], wvp, wet2,
                           bq[l].reshape(1, D), bke, bvp)
        (acc,) = _edge_call(q2, kv, ea, srcb, dstb)
        h2pre, ssum, ssq = _post_call(
            acc, h, We[l], Wskip[l], bskip[l].reshape(1, D),
            Wbeta[l], Wt[l], bt[l].reshape(1, D))
        h = _bn_call(h2pre, ssum, ssq,
                     bn_gamma[l].reshape(1, D), bn_beta[l].reshape(1, D))
        locs.append(h)

    z, _, _ = _pool_call(
        locs[0], locs[1], bi, Wg, bg.reshape(1, 1), W1, b1.reshape(1, D // 2),
        W2, b2.reshape(1, 1))
    return z
